# baseline (speedup 1.0000x reference)
import functools

import jax
import jax.numpy as jnp
from jax import lax
from jax.experimental import pallas as pl
from jax.experimental.pallas import tpu as pltpu
from jax.experimental.pallas import tpu_sc as plsc

D_MODEL = 64
SCALE = 8.0
_L = 16
_NC = 2
_NS = 16
_NW = _NC * _NS
_CHUNK = 128


@functools.lru_cache(maxsize=None)
def _make_kernel(B, V):
    bpw = B // _NW
    nchunk = bpw // _CHUNK
    mesh = plsc.VectorSubcoreMesh(core_axis_name="c", subcore_axis_name="s")

    @functools.partial(
        pl.kernel,
        mesh=mesh,
        out_type=jax.ShapeDtypeStruct((B, D_MODEL), jnp.float32),
        scratch_types=[
            pltpu.VMEM((nchunk, _CHUNK), jnp.int32),
            pltpu.VMEM((_CHUNK, D_MODEL), jnp.float32),
            pltpu.SemaphoreType.DMA,
        ],
        compiler_params=pltpu.CompilerParams(use_tc_tiling_on_sc=False),
    )
    def k(x_hbm, lut_hbm, out_hbm, idx_v, rows_v, sem):
        wid = lax.axis_index("s") * _NC + lax.axis_index("c")
        pltpu.sync_copy(x_hbm.at[wid], idx_v)

        def chunk_body(c, carry):
            pltpu.async_copy(lut_hbm.at[idx_v.at[c]], rows_v, sem).wait()

            def scale_body(r, carry2):
                for j in range(D_MODEL // _L):
                    sl = pl.ds(j * _L, _L)
                    rows_v[r, sl] = rows_v[r, sl] * SCALE
                return carry2

            lax.fori_loop(0, _CHUNK, scale_body, 0)
            pltpu.sync_copy(
                rows_v, out_hbm.at[pl.ds(wid * bpw + c * _CHUNK, _CHUNK)]
            )
            return carry

        lax.fori_loop(0, nchunk, chunk_body, 0)

    return k


def kernel(x, lut):
    B = x.shape[0] * x.shape[1]
    xf = x.reshape(_NW, B // _NW // _CHUNK, _CHUNK).astype(jnp.int32)
    out = _make_kernel(B, lut.shape[0])(xf, lut)
    return out.reshape(x.shape[0], x.shape[1], D_MODEL)
